# Initial kernel scaffold; baseline (speedup 1.0000x reference)
import jax
import jax.numpy as jnp
from jax.experimental import pallas as pl


def kernel(x, params, edge_index):
    raise NotImplementedError("write your pallas kernel here")


# speedup vs baseline: 14.9432x; 14.9432x over previous
import functools

import jax
import jax.numpy as jnp
from jax import lax
from jax.experimental import pallas as pl
from jax.experimental.pallas import tpu as pltpu
from jax.experimental.pallas import tpu_sc as plsc

EPSV = 1e-5
CHUNK = 80


def _make_sc_kernels(N, H, E, NC, NS):
    NW = NC * NS
    EPW = E // NW
    NCH = EPW // CHUNK
    BIG = ((N + NS - 1) // NS + 7) // 8 * 8
    NFULL = N // BIG
    REM = N - NFULL * BIG
    mesh = plsc.VectorSubcoreMesh(core_axis_name="c", subcore_axis_name="s")

    def _dist_rows(s, copy_fn):
        @pl.when(s < NFULL)
        def _():
            copy_fn(pl.multiple_of(s * BIG, 8), BIG)

        if REM:
            @pl.when(s == NFULL)
            def _():
                copy_fn(NFULL * BIG, REM)

    @functools.partial(
        pl.kernel,
        out_type=jax.ShapeDtypeStruct((NC, N), jnp.float32),
        mesh=mesh,
        scratch_types=[
            pltpu.VMEM((NCH, CHUNK), jnp.int32),
            pltpu.VMEM((CHUNK,), jnp.float32),
            pltpu.VMEM_SHARED((N,), jnp.float32),
        ],
    )
    def deg_kernel(dst_hbm, zeros_hbm, out_hbm, dst_v, ones_v, shared_deg):
        c = lax.axis_index("c")
        s = lax.axis_index("s")
        wid = c * NS + s
        pltpu.sync_copy(dst_hbm.at[wid], dst_v)
        for k in range(CHUNK // 16):
            ones_v[pl.ds(k * 16, 16)] = jnp.ones((16,), jnp.float32)

        @pl.when(s == 0)
        def _():
            pltpu.sync_copy(zeros_hbm, shared_deg)

        plsc.subcore_barrier()

        def body(j, carry):
            pltpu.sync_copy(ones_v, shared_deg.at[dst_v.at[j]], add=True)
            return carry

        lax.fori_loop(0, NCH, body, 0)
        plsc.subcore_barrier()

        @pl.when(s == 0)
        def _():
            pltpu.sync_copy(shared_deg, out_hbm.at[c])

    @functools.partial(
        pl.kernel,
        out_type=jax.ShapeDtypeStruct((NC, N, H), jnp.float32),
        mesh=mesh,
        scratch_types=[
            pltpu.VMEM((NCH, CHUNK), jnp.int32),
            pltpu.VMEM((NCH, CHUNK), jnp.int32),
            pltpu.VMEM((CHUNK, H), jnp.float32),
            pltpu.VMEM_SHARED((N, H), jnp.float32),
            pltpu.SemaphoreType.DMA,
        ],
    )
    def agg_kernel(y_hbm, src_hbm, dst_hbm, zeros_hbm, out_hbm,
                   src_v, dst_v, rows, acc, sem):
        c = lax.axis_index("c")
        s = lax.axis_index("s")
        wid = c * NS + s
        pltpu.sync_copy(src_hbm.at[wid], src_v)
        pltpu.sync_copy(dst_hbm.at[wid], dst_v)

        def _zero(off, size):
            pltpu.sync_copy(zeros_hbm.at[pl.ds(off, size)],
                            acc.at[pl.ds(off, size)])

        _dist_rows(s, _zero)
        plsc.subcore_barrier()

        def body(j, carry):
            pltpu.async_copy(y_hbm.at[src_v.at[j]], rows, sem).wait()
            pltpu.sync_copy(rows, acc.at[dst_v.at[j]], add=True)
            return carry

        lax.fori_loop(0, NCH, body, 0)
        plsc.subcore_barrier()

        def _writeout(off, size):
            pltpu.sync_copy(acc.at[pl.ds(off, size)],
                            out_hbm.at[c, pl.ds(off, size)])

        _dist_rows(s, _writeout)

    return deg_kernel, agg_kernel


def _bn_relu(v, g, b):
    mu = jnp.mean(v, axis=0, keepdims=True)
    var = jnp.mean(v * v, axis=0, keepdims=True) - mu * mu
    return jnp.maximum(g * (v - mu) * lax.rsqrt(var + EPSV) + b, 0.0)


def _dinv_of(degt):
    deg = degt[:, 0:1] + degt[:, 1:2] + 1.0
    return lax.rsqrt(deg)


def _pre_body(x_ref, degt_ref, g0, b0, W, y_ref):
    dinv = _dinv_of(degt_ref[...])
    h = _bn_relu(x_ref[...], g0[...], b0[...])
    y_ref[...] = jnp.dot(h, W[...], preferred_element_type=jnp.float32) * dinv


def _post_body(res_ref, y_ref, p0, p1, degt, bconv, g1, b1, W1, bias1,
               g2, b2, g0n, b0n, Wn, xn_ref, yn_ref):
    dinv = _dinv_of(degt[...])
    out = dinv * (p0[...] + p1[...] + y_ref[...]) + bconv[...]
    h = _bn_relu(out, g1[...], b1[...])
    z = jnp.dot(h, W1[...], preferred_element_type=jnp.float32) + bias1[...]
    xn = _bn_relu(z, g2[...], b2[...]) + res_ref[...]
    xn_ref[...] = xn
    h0 = _bn_relu(xn, g0n[...], b0n[...])
    yn_ref[...] = jnp.dot(h0, Wn[...], preferred_element_type=jnp.float32) * dinv


def _last_body(res_ref, y_ref, p0, p1, degt, bconv, g1, b1, W1, bias1,
               g2, b2, xn_ref):
    dinv = _dinv_of(degt[...])
    out = dinv * (p0[...] + p1[...] + y_ref[...]) + bconv[...]
    h = _bn_relu(out, g1[...], b1[...])
    z = jnp.dot(h, W1[...], preferred_element_type=jnp.float32) + bias1[...]
    xn_ref[...] = _bn_relu(z, g2[...], b2[...]) + res_ref[...]


def kernel(x, params, edge_index):
    N, H = x.shape
    E = edge_index.shape[1]
    info = plsc.get_sparse_core_info()
    NC, NS = info.num_cores, info.num_subcores
    NW = NC * NS
    EPW = E // NW
    NCH = EPW // CHUNK

    deg_kernel, agg_kernel = _make_sc_kernels(N, H, E, NC, NS)

    src3 = edge_index[0].reshape(NW, NCH, CHUNK)
    dst3 = edge_index[1].reshape(NW, NCH, CHUNK)
    zeros_n = jnp.zeros((N,), jnp.float32)
    zeros_nh = jnp.zeros((N, H), jnp.float32)

    degp = deg_kernel(dst3, zeros_n)
    degt = degp.T

    f32 = jnp.float32
    nh = jax.ShapeDtypeStruct((N, H), f32)

    def r1(v):
        return v.reshape(1, H)

    L0 = params[0]
    y = pl.pallas_call(_pre_body, out_shape=nh)(
        x, degt, r1(L0['bn0_g']), r1(L0['bn0_b']), L0['W'])

    xcur = x
    nlayers = len(params)
    for l in range(nlayers):
        p = agg_kernel(y, src3, dst3, zeros_nh)
        Ll = params[l]
        if l + 1 < nlayers:
            Ln = params[l + 1]
            xcur, y = pl.pallas_call(_post_body, out_shape=(nh, nh))(
                xcur, y, p[0], p[1], degt,
                r1(Ll['b']), r1(Ll['bn1_g']), r1(Ll['bn1_b']),
                Ll['W1'], r1(Ll['b1']), r1(Ll['bn2_g']), r1(Ll['bn2_b']),
                r1(Ln['bn0_g']), r1(Ln['bn0_b']), Ln['W'])
        else:
            xcur = pl.pallas_call(_last_body, out_shape=nh)(
                xcur, y, p[0], p[1], degt,
                r1(Ll['b']), r1(Ll['bn1_g']), r1(Ll['bn1_b']),
                Ll['W1'], r1(Ll['b1']), r1(Ll['bn2_g']), r1(Ll['bn2_b']))
    return xcur
